# Initial kernel scaffold; baseline (speedup 1.0000x reference)
import functools

import jax
import jax.numpy as jnp
from jax.experimental import pallas as pl
from jax.experimental.pallas import tpu as pltpu

_NUM_CONTEXTS = 32
_NUM_GENS = 96
_MAX_DRAFT = 3
_ROWS = _NUM_CONTEXTS + _NUM_GENS * (_MAX_DRAFT + 1)
_VOCAB = 100000
_VB = 2048
_NB = -(-_VOCAB // _VB)


def _argmax_body(x_ref, draft_ref, tt_ref, val_ref, acc_ref, m_scr, a_scr):
    j = pl.program_id(0)
    x = x_ref[...]
    col = jax.lax.broadcasted_iota(jnp.int32, (_ROWS, _VB), 1) + j * _VB
    x = jnp.where(col < _VOCAB, x, -jnp.inf)
    lmax = jnp.max(x, axis=1, keepdims=True)
    larg = jnp.min(jnp.where(x == lmax, col, _VOCAB), axis=1, keepdims=True)

    @pl.when(j == 0)
    def _init():
        m_scr[...] = lmax
        a_scr[...] = larg

    @pl.when(j > 0)
    def _acc():
        better = lmax > m_scr[...]
        m_scr[...] = jnp.where(better, lmax, m_scr[...])
        a_scr[...] = jnp.where(better, larg, a_scr[...])

    @pl.when(j == _NB - 1)
    def _fin():
        tt = a_scr[...]
        tt_ref[...] = tt
        val_ref[...] = m_scr[...]
        gen_t = tt[_NUM_CONTEXTS:, 0].reshape(_NUM_GENS, _MAX_DRAFT + 1)
        draft = draft_ref[...]
        m = (draft == gen_t[:, :_MAX_DRAFT]).astype(jnp.int32)
        cum = jnp.cumprod(m, axis=1)
        acc_ref[...] = 1 + jnp.sum(cum, axis=1, keepdims=True)


@jax.jit
def kernel(logits, draft_tokens):
    if logits.ndim == 1:
        logits = logits[None, :]
    draft_tokens = draft_tokens.astype(jnp.int32)
    tt, vals, num_acc_gen = pl.pallas_call(
        _argmax_body,
        grid=(_NB,),
        in_specs=[
            pl.BlockSpec((_ROWS, _VB), lambda j: (0, j)),
            pl.BlockSpec((_NUM_GENS, _MAX_DRAFT), lambda j: (0, 0)),
        ],
        out_specs=[
            pl.BlockSpec((_ROWS, 1), lambda j: (0, 0)),
            pl.BlockSpec((_ROWS, 1), lambda j: (0, 0)),
            pl.BlockSpec((_NUM_GENS, 1), lambda j: (0, 0)),
        ],
        out_shape=[
            jax.ShapeDtypeStruct((_ROWS, 1), jnp.int32),
            jax.ShapeDtypeStruct((_ROWS, 1), jnp.float32),
            jax.ShapeDtypeStruct((_NUM_GENS, 1), jnp.int32),
        ],
        scratch_shapes=[
            pltpu.VMEM((_ROWS, 1), jnp.float32),
            pltpu.VMEM((_ROWS, 1), jnp.int32),
        ],
    )(logits, draft_tokens)

    target_tokens = tt[:, 0]
    accepted_values = vals[:, 0]
    ctx_accepted = jnp.concatenate(
        [target_tokens[:_NUM_CONTEXTS, None],
         jnp.zeros((_NUM_CONTEXTS, _MAX_DRAFT), dtype=jnp.int32)], axis=1)
    gen_accepted = target_tokens[_NUM_CONTEXTS:].reshape(_NUM_GENS, _MAX_DRAFT + 1)
    accepted_tokens = jnp.concatenate([ctx_accepted, gen_accepted], axis=0)
    num_accepted = jnp.concatenate(
        [jnp.ones((_NUM_CONTEXTS,), dtype=jnp.int32), num_acc_gen[:, 0]], axis=0)
    return accepted_tokens, num_accepted, accepted_values


# speedup vs baseline: 1.2704x; 1.2704x over previous
import functools

import jax
import jax.numpy as jnp
from jax.experimental import pallas as pl
from jax.experimental.pallas import tpu as pltpu

_NUM_CONTEXTS = 32
_NUM_GENS = 96
_MAX_DRAFT = 3
_ROWS = _NUM_CONTEXTS + _NUM_GENS * (_MAX_DRAFT + 1)
_VOCAB = 100000
_VB = 2048
_NB = -(-_VOCAB // _VB)


def _argmax_body(x_ref, draft_ref, tt_ref, val_ref, acc_ref, m_scr, a_scr):
    j = pl.program_id(0)
    x = x_ref[...]
    col = jax.lax.broadcasted_iota(jnp.int32, (_ROWS, _VB), 1) + j * _VB
    x = jnp.where(col < _VOCAB, x, -jnp.inf)
    lmax = jnp.max(x, axis=1, keepdims=True)
    larg = jnp.min(jnp.where(x == lmax, col, _VOCAB), axis=1, keepdims=True)

    @pl.when(j == 0)
    def _init():
        m_scr[...] = lmax
        a_scr[...] = larg

    @pl.when(j > 0)
    def _acc():
        better = lmax > m_scr[...]
        m_scr[...] = jnp.where(better, lmax, m_scr[...])
        a_scr[...] = jnp.where(better, larg, a_scr[...])

    @pl.when(j == _NB - 1)
    def _fin():
        tt = a_scr[...]
        tt_ref[...] = tt
        val_ref[...] = m_scr[...]
        gen_t = tt[_NUM_CONTEXTS:, 0].reshape(_NUM_GENS, _MAX_DRAFT + 1)
        draft = draft_ref[...]
        m = (draft == gen_t[:, :_MAX_DRAFT]).astype(jnp.int32)
        run = m[:, 0:1]
        total = run
        for k in range(1, _MAX_DRAFT):
            run = run * m[:, k:k + 1]
            total = total + run
        acc_ref[...] = 1 + total


@jax.jit
def kernel(logits, draft_tokens):
    if logits.ndim == 1:
        logits = logits[None, :]
    draft_tokens = draft_tokens.astype(jnp.int32)
    tt, vals, num_acc_gen = pl.pallas_call(
        _argmax_body,
        grid=(_NB,),
        in_specs=[
            pl.BlockSpec((_ROWS, _VB), lambda j: (0, j)),
            pl.BlockSpec((_NUM_GENS, _MAX_DRAFT), lambda j: (0, 0)),
        ],
        out_specs=[
            pl.BlockSpec((_ROWS, 1), lambda j: (0, 0)),
            pl.BlockSpec((_ROWS, 1), lambda j: (0, 0)),
            pl.BlockSpec((_NUM_GENS, 1), lambda j: (0, 0)),
        ],
        out_shape=[
            jax.ShapeDtypeStruct((_ROWS, 1), jnp.int32),
            jax.ShapeDtypeStruct((_ROWS, 1), jnp.float32),
            jax.ShapeDtypeStruct((_NUM_GENS, 1), jnp.int32),
        ],
        scratch_shapes=[
            pltpu.VMEM((_ROWS, 1), jnp.float32),
            pltpu.VMEM((_ROWS, 1), jnp.int32),
        ],
    )(logits, draft_tokens)

    target_tokens = tt[:, 0]
    accepted_values = vals[:, 0]
    ctx_accepted = jnp.concatenate(
        [target_tokens[:_NUM_CONTEXTS, None],
         jnp.zeros((_NUM_CONTEXTS, _MAX_DRAFT), dtype=jnp.int32)], axis=1)
    gen_accepted = target_tokens[_NUM_CONTEXTS:].reshape(_NUM_GENS, _MAX_DRAFT + 1)
    accepted_tokens = jnp.concatenate([ctx_accepted, gen_accepted], axis=0)
    num_accepted = jnp.concatenate(
        [jnp.ones((_NUM_CONTEXTS,), dtype=jnp.int32), num_acc_gen[:, 0]], axis=0)
    return accepted_tokens, num_accepted, accepted_values


# speedup vs baseline: 1.3244x; 1.0425x over previous
import functools

import jax
import jax.numpy as jnp
from jax.experimental import pallas as pl
from jax.experimental.pallas import tpu as pltpu

_NUM_CONTEXTS = 32
_NUM_GENS = 96
_MAX_DRAFT = 3
_ROWS = _NUM_CONTEXTS + _NUM_GENS * (_MAX_DRAFT + 1)
_VOCAB = 100000
_VB = 2048
_NB = -(-_VOCAB // _VB)
_TAIL = _VOCAB - (_NB - 1) * _VB


def _argmax_body(x_ref, draft_ref, tt_ref, val_ref, acc_ref, m_scr, a_scr):
    j = pl.program_id(0)

    def _reduce(x):
        col = jax.lax.broadcasted_iota(jnp.int32, (_ROWS, _VB), 1)
        lmax = jnp.max(x, axis=1, keepdims=True)
        larg = jnp.min(jnp.where(x == lmax, col, _VB), axis=1, keepdims=True)
        return lmax, larg + j * _VB

    def _accum(lmax, larg):
        better = lmax > m_scr[...]
        m_scr[...] = jnp.where(better, lmax, m_scr[...])
        a_scr[...] = jnp.where(better, larg, a_scr[...])

    @pl.when(j == 0)
    def _init():
        lmax, larg = _reduce(x_ref[...])
        m_scr[...] = lmax
        a_scr[...] = larg

    @pl.when((j > 0) & (j < _NB - 1))
    def _mid():
        _accum(*_reduce(x_ref[...]))

    @pl.when(j == _NB - 1)
    def _fin():
        col = jax.lax.broadcasted_iota(jnp.int32, (_ROWS, _VB), 1)
        x = jnp.where(col < _TAIL, x_ref[...], -jnp.inf)
        _accum(*_reduce(x))
        tt = a_scr[...]
        tt_ref[...] = tt
        val_ref[...] = m_scr[...]
        gen_t = tt[_NUM_CONTEXTS:, 0].reshape(_NUM_GENS, _MAX_DRAFT + 1)
        draft = draft_ref[...]
        m = (draft == gen_t[:, :_MAX_DRAFT]).astype(jnp.int32)
        run = m[:, 0:1]
        total = run
        for k in range(1, _MAX_DRAFT):
            run = run * m[:, k:k + 1]
            total = total + run
        acc_ref[...] = 1 + total


@jax.jit
def kernel(logits, draft_tokens):
    if logits.ndim == 1:
        logits = logits[None, :]
    draft_tokens = draft_tokens.astype(jnp.int32)
    tt, vals, num_acc_gen = pl.pallas_call(
        _argmax_body,
        grid=(_NB,),
        in_specs=[
            pl.BlockSpec((_ROWS, _VB), lambda j: (0, j)),
            pl.BlockSpec((_NUM_GENS, _MAX_DRAFT), lambda j: (0, 0)),
        ],
        out_specs=[
            pl.BlockSpec((_ROWS, 1), lambda j: (0, 0)),
            pl.BlockSpec((_ROWS, 1), lambda j: (0, 0)),
            pl.BlockSpec((_NUM_GENS, 1), lambda j: (0, 0)),
        ],
        out_shape=[
            jax.ShapeDtypeStruct((_ROWS, 1), jnp.int32),
            jax.ShapeDtypeStruct((_ROWS, 1), jnp.float32),
            jax.ShapeDtypeStruct((_NUM_GENS, 1), jnp.int32),
        ],
        scratch_shapes=[
            pltpu.VMEM((_ROWS, 1), jnp.float32),
            pltpu.VMEM((_ROWS, 1), jnp.int32),
        ],
    )(logits, draft_tokens)

    target_tokens = tt[:, 0]
    accepted_values = vals[:, 0]
    ctx_accepted = jnp.concatenate(
        [target_tokens[:_NUM_CONTEXTS, None],
         jnp.zeros((_NUM_CONTEXTS, _MAX_DRAFT), dtype=jnp.int32)], axis=1)
    gen_accepted = target_tokens[_NUM_CONTEXTS:].reshape(_NUM_GENS, _MAX_DRAFT + 1)
    accepted_tokens = jnp.concatenate([ctx_accepted, gen_accepted], axis=0)
    num_accepted = jnp.concatenate(
        [jnp.ones((_NUM_CONTEXTS,), dtype=jnp.int32), num_acc_gen[:, 0]], axis=0)
    return accepted_tokens, num_accepted, accepted_values
